# baseline (speedup 1.0000x reference)
import functools

import jax
import jax.numpy as jnp
from jax import lax
from jax.experimental import pallas as pl
from jax.experimental.pallas import tpu as pltpu
from jax.experimental.pallas import tpu_sc as plsc

B, S, D = 1024, 200, 64
VOCAB2 = 500000
NC, NS = 2, 16
NW = NC * NS
LANES = 16
BBLK = 128
NBLK = B // BBLK
NBLOCKS = S * NBLK
BLK_PER_W = NBLOCKS // NW
PCOLS = 256


def _emb_body(x_hbm, tok_hbm, pos_hbm, out_hbm,
              idx_v, j_v, cb_v, rows_v, ob_v, pos_v, sem):
    wid = lax.axis_index("s") * NC + lax.axis_index("c")
    t0 = wid * BLK_PER_W

    pltpu.sync_copy(pos_hbm.at[:, pl.ds(0, PCOLS)], pos_v)

    def block_body(bi, carry):
        t = t0 + bi
        s = t // NBLK
        b0 = (t % NBLK) * BBLK

        pltpu.sync_copy(x_hbm.at[s, pl.ds(b0, BBLK)], idx_v)

        for g in range(BBLK // LANES):
            sl = pl.ds(g * LANES, LANES)
            v = idx_v[sl]
            j_v[sl] = lax.shift_right_logical(v, 1)
            cb_v[sl] = lax.shift_left(lax.bitwise_and(v, 1), 6)

        pltpu.async_copy(tok_hbm.at[j_v], rows_v, sem).wait()

        iota = jax.lax.iota(jnp.int32, LANES)
        rowids = [iota + (g * LANES) for g in range(BBLK // LANES)]
        cbs = [cb_v[pl.ds(g * LANES, LANES)] for g in range(BBLK // LANES)]
        sv = jnp.full((LANES,), s, jnp.int32)

        def d_body(d, carry):
            rids, cbl = carry
            dv = jnp.full((LANES,), d, jnp.int32)
            pos_vec = plsc.load_gather(pos_v, [dv, sv])
            for g in range(BBLK // LANES):
                vals = plsc.load_gather(rows_v, [rids[g], cbl[g] + d])
                ob_v[d, pl.ds(g * LANES, LANES)] = vals + pos_vec
            return carry

        lax.fori_loop(0, D, d_body, (tuple(rowids), tuple(cbs)))

        pltpu.sync_copy(ob_v, out_hbm.at[s, :, pl.ds(b0, BBLK)])
        return carry

    lax.fori_loop(0, BLK_PER_W, block_body, 0)


@functools.partial(
    pl.kernel,
    out_type=jax.ShapeDtypeStruct((S, D, B), jnp.float32),
    mesh=plsc.VectorSubcoreMesh(core_axis_name="c", subcore_axis_name="s"),
    scratch_types=[
        pltpu.VMEM((BBLK,), jnp.int32),
        pltpu.VMEM((BBLK,), jnp.int32),
        pltpu.VMEM((BBLK,), jnp.int32),
        pltpu.VMEM((BBLK, 128), jnp.float32),
        pltpu.VMEM((D, BBLK), jnp.float32),
        pltpu.VMEM((D, PCOLS), jnp.float32),
        pltpu.SemaphoreType.DMA,
    ],
    compiler_params=pltpu.CompilerParams(
        use_tc_tiling_on_sc=True, needs_layout_passes=False
    ),
)
def _emb(x_hbm, tok_hbm, pos_hbm, out_hbm, *scratch):
    _emb_body(x_hbm, tok_hbm, pos_hbm, out_hbm, *scratch)


@jax.jit
def kernel(x, token_table, pos_table):
    xT = x.astype(jnp.int32).T
    tok2 = token_table.reshape(VOCAB2, 128)
    posT = pos_table.T
    out_t = _emb(xT, tok2, posT)
    return out_t.transpose(2, 0, 1)
